# Initial kernel scaffold; baseline (speedup 1.0000x reference)
import jax
import jax.numpy as jnp
from jax.experimental import pallas as pl


def kernel(queries, keys, values):
    raise NotImplementedError("write your pallas kernel here")


# speedup vs baseline: 1.5670x; 1.5670x over previous
import functools

import jax
import jax.numpy as jnp
from jax import lax
from jax.experimental import pallas as pl
from jax.experimental.pallas import tpu as pltpu
from jax.experimental.pallas import tpu_sc as plsc

import numpy as np

_FACTOR = 5
_NEG = np.float32(-3.0e38)


def _stage1_body(q_ref, ks_ref, m_ref, *, seq_len):
    qb = q_ref[0]
    ksb = ks_ref[0]
    s = lax.dot_general(qb, ksb, (((1,), (1,)), ((), ())),
                        preferred_element_type=jnp.float32)
    m_ref[0, 0] = jnp.max(s, axis=1) - jnp.sum(s, axis=1) * (1.0 / seq_len)


def _stage1(q, ks):
    P, L, D = q.shape
    U = ks.shape[1]
    return pl.pallas_call(
        functools.partial(_stage1_body, seq_len=L),
        grid=(P,),
        in_specs=[
            pl.BlockSpec((1, L, D), lambda i: (i, 0, 0)),
            pl.BlockSpec((1, U, D), lambda i: (i, 0, 0)),
        ],
        out_specs=pl.BlockSpec((1, 1, L), lambda i: (i, 0, 0)),
        out_shape=jax.ShapeDtypeStruct((P, 1, L), jnp.float32),
    )(q, ks).reshape(P, L)


def _lane0_mask():
    return jnp.arange(16, dtype=jnp.int32) == 0


def _store_scalar(ref, pos, val):
    idx = jnp.full((16,), pos, dtype=jnp.int32)
    x = jnp.full((16,), val, dtype=ref.dtype)
    plsc.store_scatter(ref, [idx], x, mask=_lane0_mask())


def _sc_body(m_hbm, qflat_hbm, idx_hbm, qred_hbm,
             row_v, cmax_v, idx_v, gidx_v, rows_v, sem,
             *, seq_len, n_top, pairs_per_subcore):
    n_chunks = seq_len // 16
    cvecs = n_chunks // 16
    upad = ((n_top + 15) // 16) * 16
    wid = lax.axis_index("s") * 2 + lax.axis_index("c")

    for p in range(pairs_per_subcore):
        pair = wid * pairs_per_subcore + p
        pltpu.sync_copy(m_hbm.at[pair], row_v)

        def _init(j, carry):
            v = row_v[pl.ds(j * 16, 16)]
            _store_scalar(cmax_v, j, jnp.max(v))
            return carry
        lax.fori_loop(0, n_chunks, _init, 0)

        for t in range(n_top // 16, 8):
            idx_v[pl.ds(t * 16, 16)] = jnp.zeros((16,), jnp.int32)

        def _extract(t, carry):
            rm = cmax_v[pl.ds(0, 16)]
            for j in range(1, cvecs):
                rm = jnp.maximum(rm, cmax_v[pl.ds(j * 16, 16)])
            gmax = jnp.max(rm)

            def _find(j, best):
                mj = cmax_v[pl.ds(j * 16, 16)] == gmax
                fj = plsc.all_reduce_ffs(mj)
                fj = jnp.asarray(fj).reshape(-1)[0]
                cand = j * 16 + fj
                hit = (fj < 16) & (best >= n_chunks)
                return jnp.where(hit, cand, best)
            chunk = lax.fori_loop(0, cvecs, _find, jnp.int32(n_chunks))

            v = row_v[pl.ds(chunk * 16, 16)]
            lane = plsc.all_reduce_ffs(v == gmax)
            lane = jnp.asarray(lane).reshape(-1)[0]
            elem = chunk * 16 + lane
            _store_scalar(idx_v, t, elem)
            v2 = jnp.where(jnp.arange(16, dtype=jnp.int32) == lane, _NEG, v)
            row_v[pl.ds(chunk * 16, 16)] = v2
            _store_scalar(cmax_v, chunk, jnp.max(v2))
            return carry
        lax.fori_loop(0, n_top, _extract, 0)

        base = pair * seq_len
        for t in range(upad // 16):
            gidx_v[pl.ds(t * 16, 16)] = lax.shift_right_logical(
                idx_v[pl.ds(t * 16, 16)] + base, 1)

        pltpu.async_copy(qflat_hbm.at[gidx_v], rows_v, sem).wait()
        pltpu.sync_copy(rows_v.at[pl.ds(0, n_top)], qred_hbm.at[pair])
        pltpu.sync_copy(idx_v, idx_hbm.at[pair])


def _sc_topk_gather(m, qflat):
    P, L = m.shape
    D = qflat.shape[1]
    U = 40
    upad = 48
    pps = P // 32
    mesh = plsc.VectorSubcoreMesh(core_axis_name="c", subcore_axis_name="s")
    fn = pl.kernel(
        functools.partial(_sc_body, seq_len=L, n_top=U, pairs_per_subcore=pps),
        out_type=[
            jax.ShapeDtypeStruct((P, 128), jnp.int32),
            jax.ShapeDtypeStruct((P, U, 128), jnp.float32),
        ],
        mesh=mesh,
        compiler_params=pltpu.CompilerParams(needs_layout_passes=False),
        scratch_types=[
            pltpu.VMEM((L,), jnp.float32),
            pltpu.VMEM((L // 16,), jnp.float32),
            pltpu.VMEM((128,), jnp.int32),
            pltpu.VMEM((upad,), jnp.int32),
            pltpu.VMEM((upad, 128), jnp.float32),
            pltpu.SemaphoreType.DMA,
        ],
    )
    return fn(m, qflat)


def _stage2_body(k_ref, v_ref, qr_ref, idx_ref, out_ref, *, scale):
    kb = k_ref[0]
    vb = v_ref[0]
    qr2 = qr_ref[0]
    idxv = idx_ref[0, 0][: qr2.shape[0]]

    par = (idxv & 1)[:, None] == 1
    qr = jnp.where(par, qr2[:, 64:128], qr2[:, 0:64])
    scores = lax.dot_general(qr, kb, (((1,), (1,)), ((), ())),
                             preferred_element_type=jnp.float32) * scale
    mx = jnp.max(scores, axis=1, keepdims=True)
    e = jnp.exp(scores - mx)
    attn = e / jnp.sum(e, axis=1, keepdims=True)
    update = lax.dot_general(attn, vb, (((1,), (0,)), ((), ())),
                             preferred_element_type=jnp.float32)
    vsum = jnp.sum(vb, axis=0)

    iota = lax.broadcasted_iota(jnp.int32, idxv.shape + kb.shape[:1], 1)
    onehot = (iota == idxv[:, None]).astype(jnp.float32)
    scat = lax.dot_general(onehot, update - vsum[None, :],
                           (((0,), (0,)), ((), ())),
                           preferred_element_type=jnp.float32)
    out_ref[0] = scat + vsum[None, :]


def _stage2(k, v, qred, idx3):
    P, L, D = k.shape
    U = qred.shape[1]
    scale = 1.0 / np.sqrt(D)
    return pl.pallas_call(
        functools.partial(_stage2_body, scale=scale),
        grid=(P,),
        in_specs=[
            pl.BlockSpec((1, L, D), lambda i: (i, 0, 0)),
            pl.BlockSpec((1, L, D), lambda i: (i, 0, 0)),
            pl.BlockSpec((1, U, 128), lambda i: (i, 0, 0)),
            pl.BlockSpec((1, 1, 128), lambda i: (i, 0, 0)),
        ],
        out_specs=pl.BlockSpec((1, L, D), lambda i: (i, 0, 0)),
        out_shape=jax.ShapeDtypeStruct((P, L, D), jnp.float32),
    )(k, v, qred, idx3)


def kernel(queries, keys, values):
    B, L, H, D = queries.shape
    S = keys.shape[1]
    P = B * H
    q = queries.reshape(P, L, 64)
    k = keys.reshape(P, S, 64)
    v = values.reshape(P, S, 64)

    u = _FACTOR * int(np.ceil(np.log(L)))
    samp = jax.random.randint(jax.random.key(42), (u,), 0, S)
    ks = jnp.take(k, samp, axis=1)

    m = _stage1(q, ks)
    idx, qred = _sc_topk_gather(m, q.reshape(P * L // 2, 128))
    ctx = _stage2(k, v, qred, idx.reshape(P, 1, 128))
    return ctx.reshape(B, H, L, 64)


# speedup vs baseline: 2.3060x; 1.4716x over previous
import functools

import jax
import jax.numpy as jnp
from jax import lax
from jax.experimental import pallas as pl
from jax.experimental.pallas import tpu as pltpu
from jax.experimental.pallas import tpu_sc as plsc

import numpy as np

_FACTOR = 5
_NEG = np.float32(-3.0e38)


def _stage1_body(qt_ref, ks_ref, m_ref, *, seq_len, n_heads):
    ksb = ks_ref[0]
    inv = 1.0 / seq_len
    for h in range(n_heads):
        qt_h = qt_ref[0, h]
        s = lax.dot_general(ksb, qt_h, (((1,), (0,)), ((), ())),
                            preferred_element_type=jnp.float32)
        m_ref[0, 0, pl.ds(h * qt_h.shape[1], qt_h.shape[1])] = (
            jnp.max(s, axis=0) - jnp.sum(s, axis=0) * inv)


def _stage1(qt, ks):
    B, HV, D, L = qt.shape
    P = B * HV
    U = ks.shape[1]
    LBLK = L // HV
    return pl.pallas_call(
        functools.partial(_stage1_body, seq_len=L, n_heads=HV),
        grid=(P,),
        in_specs=[
            pl.BlockSpec((1, HV, D, LBLK), lambda i: (i // 16, 0, 0, i % 16)),
            pl.BlockSpec((1, U, D), lambda i: (i, 0, 0)),
        ],
        out_specs=pl.BlockSpec((1, 1, L), lambda i: (i, 0, 0)),
        out_shape=jax.ShapeDtypeStruct((P, 1, L), jnp.float32),
    )(qt, ks)


def _lane0_mask():
    return jnp.arange(16, dtype=jnp.int32) == 0


def _store_scalar(ref, pos, val):
    idx = jnp.full((16,), pos, dtype=jnp.int32)
    x = jnp.full((16,), val, dtype=ref.dtype)
    plsc.store_scatter(ref, [idx], x, mask=_lane0_mask())


def _sc_body(m_hbm, idx_hbm, row_v, cmax_v, idx_v,
             *, seq_len, n_top, lblk, pairs_per_subcore):
    n_chunks = seq_len // 16
    cvecs = n_chunks // 16
    wid = lax.axis_index("s") * 2 + lax.axis_index("c")

    for p in range(pairs_per_subcore):
        pair = wid * pairs_per_subcore + p
        pltpu.sync_copy(m_hbm.at[pair, 0], row_v)

        def _init(j, carry):
            v = row_v[pl.ds(j * 16, 16)]
            _store_scalar(cmax_v, j, jnp.max(v))
            return carry
        lax.fori_loop(0, n_chunks, _init, 0)

        for t in range(n_top // 16, 8):
            idx_v[pl.ds(t * 16, 16)] = jnp.zeros((16,), jnp.int32)

        def _extract(t, carry):
            rm = cmax_v[pl.ds(0, 16)]
            for j in range(1, cvecs):
                rm = jnp.maximum(rm, cmax_v[pl.ds(j * 16, 16)])
            gmax = jnp.max(rm)

            def _find(j, best):
                mj = cmax_v[pl.ds(j * 16, 16)] == gmax
                fj = plsc.all_reduce_ffs(mj)
                fj = jnp.asarray(fj).reshape(-1)[0]
                cand = j * 16 + fj
                hit = (fj < 16) & (best >= n_chunks)
                return jnp.where(hit, cand, best)
            chunk = lax.fori_loop(0, cvecs, _find, jnp.int32(n_chunks))

            v = row_v[pl.ds(chunk * 16, 16)]
            lane = plsc.all_reduce_ffs(v == gmax)
            lane = jnp.asarray(lane).reshape(-1)[0]
            elem = chunk * 16 + lane
            vrow = (elem % lblk) * (seq_len // lblk) + elem // lblk
            _store_scalar(idx_v, t, vrow)
            v2 = jnp.where(jnp.arange(16, dtype=jnp.int32) == lane, _NEG, v)
            row_v[pl.ds(chunk * 16, 16)] = v2
            _store_scalar(cmax_v, chunk, jnp.max(v2))
            return carry
        lax.fori_loop(0, n_top, _extract, 0)

        pltpu.sync_copy(idx_v, idx_hbm.at[pair])


def _sc_topk(m, lblk):
    P, _, L = m.shape
    U = 40
    pps = P // 32
    mesh = plsc.VectorSubcoreMesh(core_axis_name="c", subcore_axis_name="s")
    fn = pl.kernel(
        functools.partial(_sc_body, seq_len=L, n_top=U, lblk=lblk,
                          pairs_per_subcore=pps),
        out_type=jax.ShapeDtypeStruct((P, 128), jnp.int32),
        mesh=mesh,
        compiler_params=pltpu.CompilerParams(needs_layout_passes=False),
        scratch_types=[
            pltpu.VMEM((L,), jnp.float32),
            pltpu.VMEM((L // 16,), jnp.float32),
            pltpu.VMEM((128,), jnp.int32),
        ],
    )
    return fn(m)


def _stage2_body(kt_ref, vt_ref, qt_ref, idx_ref, out_ref, *, scale, n_top):
    HV = kt_ref.shape[1]
    D = kt_ref.shape[2]
    LBLK = kt_ref.shape[3]
    L = HV * LBLK
    idxv = idx_ref[0, 0][:n_top]
    hh = idxv % HV
    ll = idxv // HV

    oh_h = (lax.broadcasted_iota(jnp.int32, (n_top, HV), 1)
            == hh[:, None]).astype(jnp.float32)
    oh_l = (lax.broadcasted_iota(jnp.int32, (n_top, LBLK), 1)
            == ll[:, None]).astype(jnp.float32)

    qr = jnp.zeros((n_top, D), jnp.float32)
    for h in range(HV):
        g = lax.dot_general(oh_l, qt_ref[0, h], (((1,), (1,)), ((), ())),
                            preferred_element_type=jnp.float32)
        qr = qr + oh_h[:, h][:, None] * g

    mx = jnp.full((n_top, 1), _NEG, jnp.float32)
    den = jnp.zeros((n_top, 1), jnp.float32)
    upd = jnp.zeros((n_top, D), jnp.float32)
    vsum = jnp.zeros((1, D), jnp.float32)
    for h in range(HV):
        kt_h = kt_ref[0, h]
        vt_h = vt_ref[0, h]
        s = lax.dot_general(qr, kt_h, (((1,), (0,)), ((), ())),
                            preferred_element_type=jnp.float32) * scale
        m_new = jnp.maximum(mx, jnp.max(s, axis=1, keepdims=True))
        alpha = jnp.exp(mx - m_new)
        e = jnp.exp(s - m_new)
        den = den * alpha + jnp.sum(e, axis=1, keepdims=True)
        upd = upd * alpha + lax.dot_general(
            e, vt_h, (((1,), (1,)), ((), ())),
            preferred_element_type=jnp.float32)
        mx = m_new
        vsum = vsum + jnp.sum(vt_h, axis=1)[None, :]
    upd = upd / den

    oh2 = (lax.broadcasted_iota(jnp.int32, (n_top, L), 1)
           == idxv[:, None]).astype(jnp.float32)
    outT = lax.dot_general(upd - vsum, oh2, (((0,), (0,)), ((), ())),
                           preferred_element_type=jnp.float32)
    out_ref[0, 0] = outT + jnp.broadcast_to(vsum.reshape(D, 1), (D, L))


def _stage2(kt, vt, qt, idx3):
    B, HV, D, L = kt.shape
    P = B * HV
    LBLK = L // HV
    U = 40
    scale = 1.0 / np.sqrt(64)
    return pl.pallas_call(
        functools.partial(_stage2_body, scale=scale, n_top=U),
        grid=(P,),
        in_specs=[
            pl.BlockSpec((1, HV, D, LBLK), lambda i: (i // 16, 0, 0, i % 16)),
            pl.BlockSpec((1, HV, D, LBLK), lambda i: (i // 16, 0, 0, i % 16)),
            pl.BlockSpec((1, HV, D, LBLK), lambda i: (i // 16, 0, 0, i % 16)),
            pl.BlockSpec((1, 1, 128), lambda i: (i, 0, 0)),
        ],
        out_specs=pl.BlockSpec((1, 1, D, L), lambda i: (i // 16, i % 16, 0, 0)),
        out_shape=jax.ShapeDtypeStruct((B, HV, D, L), jnp.float32),
    )(kt, vt, qt, idx3)


def kernel(queries, keys, values):
    B, L, H, D = queries.shape
    S = keys.shape[1]
    HV = H * D // 64
    LBLK = L // HV
    P = B * HV

    qt = jnp.transpose(queries, (0, 2, 3, 1))
    kt = jnp.transpose(keys, (0, 2, 3, 1))
    vt = jnp.transpose(values, (0, 2, 3, 1))

    u = _FACTOR * int(np.ceil(np.log(L)))
    samp = jax.random.randint(jax.random.key(42), (u,), 0, S)
    sh = samp % HV
    sl = samp // HV
    lidx = (jnp.arange(HV, dtype=samp.dtype) * LBLK)[:, None] + sl[None, :]
    ks = kt[:, sh[None, :], :, lidx]
    ks = jnp.transpose(ks, (2, 0, 1, 3)).reshape(P, u, 64)

    m = _stage1(qt, ks)
    idx = _sc_topk(m, LBLK)
    ctx = _stage2(kt, vt, qt, idx.reshape(P, 1, 128))
    return jnp.transpose(ctx, (0, 1, 3, 2))
